# Initial kernel scaffold; baseline (speedup 1.0000x reference)
import jax
import jax.numpy as jnp
from jax.experimental import pallas as pl


def kernel(seq, embed, ff1_w, ff1_b, ff2_w, ff2_b, ln_g, ln_b, fc1_w, fc1_b, fc2_w, fc2_b, out_w, out_b):
    raise NotImplementedError("write your pallas kernel here")


# speedup vs baseline: 32.3133x; 32.3133x over previous
import functools

import jax
import jax.numpy as jnp
from jax import lax
from jax.experimental import pallas as pl
from jax.experimental.pallas import tpu as pltpu

_H = 32
_INNER = 8
_VOCAB = 64
_LR = 0.01
_THR = 4.0
_EPS = 1e-5
_S = 128
_UNROLL = 4


def _split3(x):
    hi = x.astype(jnp.bfloat16).astype(jnp.float32)
    r = x - hi
    mid = r.astype(jnp.bfloat16).astype(jnp.float32)
    lo = r - mid
    return hi, mid, lo


def _fw_kernel(seq_ref, embedT_ref, ff1w_ref, ff1b_ref, ff2w_ref, ff2b_ref,
               lng_ref, lnb_ref, fc1w_ref, fc1b_ref, fc2wT_ref, fc2b_ref,
               outw_ref, outb_ref,
               logits_ref, wcnt_ref,
               tab_ref, w1_ref, w2t_ref, b1_ref, b2_ref,
               ssum_ref, cnt_ref, wsum_ref, *, n_steps):
    eT = embedT_ref[...]
    zT = jnp.maximum(
        jnp.dot(ff1w_ref[...], eT, preferred_element_type=jnp.float32)
        + ff1b_ref[...], 0.0)
    fT = jnp.dot(ff2w_ref[...], zT, preferred_element_type=jnp.float32) \
        + ff2b_ref[...]
    xT = eT + fT
    mu = jnp.mean(xT, axis=0, keepdims=True)
    var = jnp.mean((xT - mu) ** 2, axis=0, keepdims=True)
    tabT = lng_ref[...] * (xT - mu) * lax.rsqrt(var + _EPS) + lnb_ref[...]
    hi, mid, lo = _split3(tabT)
    tab_ref[...] = jnp.concatenate([hi, mid, lo], axis=0)

    w1_ref[...] = jnp.broadcast_to(fc1w_ref[...][:, :, None], (_INNER, _H, _S))
    w2t_ref[...] = jnp.broadcast_to(fc2wT_ref[...][:, :, None], (_INNER, _H, _S))
    b1_ref[...] = jnp.broadcast_to(fc1b_ref[...], (_INNER, _S))
    b2_ref[...] = jnp.broadcast_to(fc2b_ref[...], (_H, _S))
    ssum_ref[...] = jnp.zeros((_H, _S), jnp.float32)
    cnt_ref[...] = jnp.zeros((1, _S), jnp.float32)
    wsum_ref[...] = jnp.zeros((1, _S), jnp.float32)

    def gather(ids):
        iota_v = lax.broadcasted_iota(jnp.int32, (_VOCAB, _S), 0)
        oh = (iota_v == ids).astype(jnp.float32)
        g = jnp.dot(tab_ref[...], oh, preferred_element_type=jnp.float32)
        return g[0:_H] + g[_H:2 * _H] + g[2 * _H:3 * _H]

    def mlp_fwd(w1, b1, w2t, b2, k):
        z1 = jnp.sum(w1 * k[None, :, :], axis=1) + b1
        a = jnp.maximum(z1, 0.0)
        pred = jnp.sum(w2t * a[:, None, :], axis=0) + b2
        return z1, a, pred

    def outer(jo, _):
        rows = seq_ref[pl.ds(_UNROLL * 2 * jo, _UNROLL * 2), :]
        w1 = w1_ref[...]
        w2t = w2t_ref[...]
        b1 = b1_ref[...]
        b2 = b2_ref[...]
        ssum = ssum_ref[...]
        cnt = cnt_ref[...]
        wsum = wsum_ref[...]
        for slot in range(_UNROLL):
            step = _UNROLL * jo + slot
            k = gather(rows[2 * slot:2 * slot + 1, :])
            v = gather(rows[2 * slot + 1:2 * slot + 2, :])
            mean_h = ssum / jnp.maximum(cnt, 1.0)
            diff = k - mean_h
            dist = jnp.sqrt(jnp.sum(diff * diff, axis=0, keepdims=True))
            write = jnp.logical_or(cnt == 0.0, dist > _THR)
            write = jnp.logical_and(write, step < n_steps)
            wf = write.astype(jnp.float32)
            z1, a, pred = mlp_fwd(w1, b1, w2t, b2, k)
            dpred = (2.0 / _H) * (pred - v)
            dz1 = jnp.sum(w2t * dpred[None, :, :], axis=1)
            dz1 = dz1 * (z1 > 0).astype(jnp.float32)
            g = wf * _LR
            gdz1 = dz1 * g
            gdpred = dpred * g
            w1 = w1 - gdz1[:, None, :] * k[None, :, :]
            b1 = b1 - gdz1
            w2t = w2t - a[:, None, :] * gdpred[None, :, :]
            b2 = b2 - gdpred
            ssum = ssum + k
            cnt = cnt + 1.0
            wsum = wsum + wf
        w1_ref[...] = w1
        w2t_ref[...] = w2t
        b1_ref[...] = b1
        b2_ref[...] = b2
        ssum_ref[...] = ssum
        cnt_ref[...] = cnt
        wsum_ref[...] = wsum
        return ()

    n_outer = (n_steps + _UNROLL - 1) // _UNROLL
    lax.fori_loop(0, n_outer, outer, ())

    L = seq_ref.shape[0]
    last = seq_ref[L - 8:L, :]
    q = gather(last[7:8, :])
    _, aq, ctx = mlp_fwd(w1_ref[...], b1_ref[...], w2t_ref[...], b2_ref[...], q)
    logits_ref[...] = jnp.dot(outw_ref[...], ctx,
                              preferred_element_type=jnp.float32) + outb_ref[...]
    wcnt_ref[...] = wsum_ref[...][None, :, :]


def kernel(seq, embed, ff1_w, ff1_b, ff2_w, ff2_b, ln_g, ln_b,
           fc1_w, fc1_b, fc2_w, fc2_b, out_w, out_b, *, interpret=False):
    B, L = seq.shape
    n = (L - 3 + 1) // 2
    nblk = B // _S
    seq_T = seq.T.astype(jnp.int32)

    logits_T, wcnt = pl.pallas_call(
        functools.partial(_fw_kernel, n_steps=n),
        out_shape=[
            jax.ShapeDtypeStruct((_VOCAB, B), jnp.float32),
            jax.ShapeDtypeStruct((nblk, 1, _S), jnp.float32),
        ],
        grid=(nblk,),
        in_specs=[
            pl.BlockSpec((L, _S), lambda i: (0, i)),
            pl.BlockSpec((_H, _VOCAB), lambda i: (0, 0)),
            pl.BlockSpec((2 * _H, _H), lambda i: (0, 0)),
            pl.BlockSpec((2 * _H, 1), lambda i: (0, 0)),
            pl.BlockSpec((_H, 2 * _H), lambda i: (0, 0)),
            pl.BlockSpec((_H, 1), lambda i: (0, 0)),
            pl.BlockSpec((_H, 1), lambda i: (0, 0)),
            pl.BlockSpec((_H, 1), lambda i: (0, 0)),
            pl.BlockSpec((_INNER, _H), lambda i: (0, 0)),
            pl.BlockSpec((_INNER, 1), lambda i: (0, 0)),
            pl.BlockSpec((_INNER, _H), lambda i: (0, 0)),
            pl.BlockSpec((_H, 1), lambda i: (0, 0)),
            pl.BlockSpec((_VOCAB, _H), lambda i: (0, 0)),
            pl.BlockSpec((_VOCAB, 1), lambda i: (0, 0)),
        ],
        out_specs=[
            pl.BlockSpec((_VOCAB, _S), lambda i: (0, i)),
            pl.BlockSpec((1, 1, _S), lambda i: (i, 0, 0)),
        ],
        scratch_shapes=[
            pltpu.VMEM((3 * _H, _VOCAB), jnp.float32),
            pltpu.VMEM((_INNER, _H, _S), jnp.float32),
            pltpu.VMEM((_INNER, _H, _S), jnp.float32),
            pltpu.VMEM((_INNER, _S), jnp.float32),
            pltpu.VMEM((_H, _S), jnp.float32),
            pltpu.VMEM((_H, _S), jnp.float32),
            pltpu.VMEM((1, _S), jnp.float32),
            pltpu.VMEM((1, _S), jnp.float32),
        ],
        compiler_params=pltpu.CompilerParams(
            dimension_semantics=("parallel",),
        ),
        name="surprise_gated_fw",
        interpret=interpret,
    )(seq_T, embed.T, ff1_w, ff1_b[:, None], ff2_w, ff2_b[:, None],
      ln_g[:, None], ln_b[:, None], fc1_w, fc1_b[:, None], fc2_w.T,
      fc2_b[:, None], out_w, out_b[:, None])

    logits = logits_T.T
    write_rate = jnp.sum(wcnt) / jnp.float32(B * n)
    return logits, write_rate


# speedup vs baseline: 38.0920x; 1.1788x over previous
import functools

import jax
import jax.numpy as jnp
from jax import lax
from jax.experimental import pallas as pl
from jax.experimental.pallas import tpu as pltpu

_H = 32
_INNER = 8
_VOCAB = 64
_LR = 0.01
_THR = 4.0
_EPS = 1e-5
_S = 256
_UNROLL = 4


def _split3(x):
    hi = x.astype(jnp.bfloat16).astype(jnp.float32)
    r = x - hi
    mid = r.astype(jnp.bfloat16).astype(jnp.float32)
    lo = r - mid
    return hi, mid, lo


def _fw_kernel(seq_ref, embedT_ref, ff1w_ref, ff1b_ref, ff2w_ref, ff2b_ref,
               lng_ref, lnb_ref, fc1w_ref, fc1b_ref, fc2wT_ref, fc2b_ref,
               outw_ref, outb_ref,
               logits_ref, wcnt_ref,
               tab_ref, w1_ref, w2t_ref, b1_ref, b2_ref,
               ssum_ref, cnt_ref, wsum_ref, *, n_steps):
    eT = embedT_ref[...]
    zT = jnp.maximum(
        jnp.dot(ff1w_ref[...], eT, preferred_element_type=jnp.float32)
        + ff1b_ref[...], 0.0)
    fT = jnp.dot(ff2w_ref[...], zT, preferred_element_type=jnp.float32) \
        + ff2b_ref[...]
    xT = eT + fT
    mu = jnp.mean(xT, axis=0, keepdims=True)
    var = jnp.mean((xT - mu) ** 2, axis=0, keepdims=True)
    tabT = lng_ref[...] * (xT - mu) * lax.rsqrt(var + _EPS) + lnb_ref[...]
    hi, mid, lo = _split3(tabT)
    tab_ref[...] = jnp.concatenate([hi, mid, lo], axis=0)

    w1_ref[...] = jnp.broadcast_to(fc1w_ref[...][:, :, None], (_INNER, _H, _S))
    w2t_ref[...] = jnp.broadcast_to(fc2wT_ref[...][:, :, None], (_INNER, _H, _S))
    b1_ref[...] = jnp.broadcast_to(fc1b_ref[...], (_INNER, _S))
    b2_ref[...] = jnp.broadcast_to(fc2b_ref[...], (_H, _S))
    ssum_ref[...] = jnp.zeros((_H, _S), jnp.float32)
    cnt_ref[...] = jnp.zeros((1, _S), jnp.float32)
    wsum_ref[...] = jnp.zeros((1, _S), jnp.float32)

    def gather(ids):
        iota_v = lax.broadcasted_iota(jnp.int32, (_VOCAB, _S), 0)
        oh = (iota_v == ids).astype(jnp.float32)
        g = jnp.dot(tab_ref[...], oh, preferred_element_type=jnp.float32)
        return g[0:_H] + g[_H:2 * _H] + g[2 * _H:3 * _H]

    def mlp_fwd(w1, b1, w2t, b2, k):
        z1 = jnp.sum(w1 * k[None, :, :], axis=1) + b1
        a = jnp.maximum(z1, 0.0)
        pred = jnp.sum(w2t * a[:, None, :], axis=0) + b2
        return z1, a, pred

    def outer(jo, _):
        rows = seq_ref[pl.ds(_UNROLL * 2 * jo, _UNROLL * 2), :]
        w1 = w1_ref[...]
        w2t = w2t_ref[...]
        b1 = b1_ref[...]
        b2 = b2_ref[...]
        ssum = ssum_ref[...]
        cnt = cnt_ref[...]
        wsum = wsum_ref[...]
        for slot in range(_UNROLL):
            step = _UNROLL * jo + slot
            k = gather(rows[2 * slot:2 * slot + 1, :])
            v = gather(rows[2 * slot + 1:2 * slot + 2, :])
            mean_h = ssum / jnp.maximum(cnt, 1.0)
            diff = k - mean_h
            dist = jnp.sqrt(jnp.sum(diff * diff, axis=0, keepdims=True))
            write = jnp.logical_or(cnt == 0.0, dist > _THR)
            write = jnp.logical_and(write, step < n_steps)
            wf = write.astype(jnp.float32)
            z1, a, pred = mlp_fwd(w1, b1, w2t, b2, k)
            dpred = (2.0 / _H) * (pred - v)
            dz1 = jnp.sum(w2t * dpred[None, :, :], axis=1)
            dz1 = dz1 * (z1 > 0).astype(jnp.float32)
            g = wf * _LR
            gdz1 = dz1 * g
            gdpred = dpred * g
            w1 = w1 - gdz1[:, None, :] * k[None, :, :]
            b1 = b1 - gdz1
            w2t = w2t - a[:, None, :] * gdpred[None, :, :]
            b2 = b2 - gdpred
            ssum = ssum + k
            cnt = cnt + 1.0
            wsum = wsum + wf
        w1_ref[...] = w1
        w2t_ref[...] = w2t
        b1_ref[...] = b1
        b2_ref[...] = b2
        ssum_ref[...] = ssum
        cnt_ref[...] = cnt
        wsum_ref[...] = wsum
        return ()

    n_outer = (n_steps + _UNROLL - 1) // _UNROLL
    lax.fori_loop(0, n_outer, outer, ())

    L = seq_ref.shape[0]
    last = seq_ref[L - 8:L, :]
    q = gather(last[7:8, :])
    _, aq, ctx = mlp_fwd(w1_ref[...], b1_ref[...], w2t_ref[...], b2_ref[...], q)
    logits_ref[...] = jnp.dot(outw_ref[...], ctx,
                              preferred_element_type=jnp.float32) + outb_ref[...]
    wcnt_ref[...] = wsum_ref[...][None, :, :]


def kernel(seq, embed, ff1_w, ff1_b, ff2_w, ff2_b, ln_g, ln_b,
           fc1_w, fc1_b, fc2_w, fc2_b, out_w, out_b, *, interpret=False):
    B, L = seq.shape
    n = (L - 3 + 1) // 2
    nblk = B // _S
    seq_T = seq.T.astype(jnp.int32)

    logits_T, wcnt = pl.pallas_call(
        functools.partial(_fw_kernel, n_steps=n),
        out_shape=[
            jax.ShapeDtypeStruct((_VOCAB, B), jnp.float32),
            jax.ShapeDtypeStruct((nblk, 1, _S), jnp.float32),
        ],
        grid=(nblk,),
        in_specs=[
            pl.BlockSpec((L, _S), lambda i: (0, i)),
            pl.BlockSpec((_H, _VOCAB), lambda i: (0, 0)),
            pl.BlockSpec((2 * _H, _H), lambda i: (0, 0)),
            pl.BlockSpec((2 * _H, 1), lambda i: (0, 0)),
            pl.BlockSpec((_H, 2 * _H), lambda i: (0, 0)),
            pl.BlockSpec((_H, 1), lambda i: (0, 0)),
            pl.BlockSpec((_H, 1), lambda i: (0, 0)),
            pl.BlockSpec((_H, 1), lambda i: (0, 0)),
            pl.BlockSpec((_INNER, _H), lambda i: (0, 0)),
            pl.BlockSpec((_INNER, 1), lambda i: (0, 0)),
            pl.BlockSpec((_INNER, _H), lambda i: (0, 0)),
            pl.BlockSpec((_H, 1), lambda i: (0, 0)),
            pl.BlockSpec((_VOCAB, _H), lambda i: (0, 0)),
            pl.BlockSpec((_VOCAB, 1), lambda i: (0, 0)),
        ],
        out_specs=[
            pl.BlockSpec((_VOCAB, _S), lambda i: (0, i)),
            pl.BlockSpec((1, 1, _S), lambda i: (i, 0, 0)),
        ],
        scratch_shapes=[
            pltpu.VMEM((3 * _H, _VOCAB), jnp.float32),
            pltpu.VMEM((_INNER, _H, _S), jnp.float32),
            pltpu.VMEM((_INNER, _H, _S), jnp.float32),
            pltpu.VMEM((_INNER, _S), jnp.float32),
            pltpu.VMEM((_H, _S), jnp.float32),
            pltpu.VMEM((_H, _S), jnp.float32),
            pltpu.VMEM((1, _S), jnp.float32),
            pltpu.VMEM((1, _S), jnp.float32),
        ],
        compiler_params=pltpu.CompilerParams(
            dimension_semantics=("arbitrary",),
        ),
        name="surprise_gated_fw",
        interpret=interpret,
    )(seq_T, embed.T, ff1_w, ff1_b[:, None], ff2_w, ff2_b[:, None],
      ln_g[:, None], ln_b[:, None], fc1_w, fc1_b[:, None], fc2_w.T,
      fc2_b[:, None], out_w, out_b[:, None])

    logits = logits_T.T
    write_rate = jnp.sum(wcnt) / jnp.float32(B * n)
    return logits, write_rate


# speedup vs baseline: 41.6946x; 1.0946x over previous
import functools

import jax
import jax.numpy as jnp
from jax import lax
from jax.experimental import pallas as pl
from jax.experimental.pallas import tpu as pltpu

_H = 32
_INNER = 8
_VOCAB = 64
_LR = 0.01
_THR = 4.0
_EPS = 1e-5
_S = 256
_UNROLL = 8


def _split3(x):
    hi = x.astype(jnp.bfloat16).astype(jnp.float32)
    r = x - hi
    mid = r.astype(jnp.bfloat16).astype(jnp.float32)
    lo = r - mid
    return hi, mid, lo


def _fw_kernel(seq_ref, embedT_ref, ff1w_ref, ff1b_ref, ff2w_ref, ff2b_ref,
               lng_ref, lnb_ref, fc1w_ref, fc1b_ref, fc2wT_ref, fc2b_ref,
               outw_ref, outb_ref,
               logits_ref, wcnt_ref,
               tab_ref, w1_ref, w2t_ref, b1_ref, b2_ref,
               ssum_ref, cnt_ref, wsum_ref, *, n_steps):
    eT = embedT_ref[...]
    zT = jnp.maximum(
        jnp.dot(ff1w_ref[...], eT, preferred_element_type=jnp.float32)
        + ff1b_ref[...], 0.0)
    fT = jnp.dot(ff2w_ref[...], zT, preferred_element_type=jnp.float32) \
        + ff2b_ref[...]
    xT = eT + fT
    mu = jnp.mean(xT, axis=0, keepdims=True)
    var = jnp.mean((xT - mu) ** 2, axis=0, keepdims=True)
    tabT = lng_ref[...] * (xT - mu) * lax.rsqrt(var + _EPS) + lnb_ref[...]
    hi, mid, lo = _split3(tabT)
    tab_ref[...] = jnp.concatenate([hi, mid, lo], axis=0)

    w1_ref[...] = jnp.broadcast_to(fc1w_ref[...][:, :, None], (_INNER, _H, _S))
    w2t_ref[...] = jnp.broadcast_to(fc2wT_ref[...][:, :, None], (_INNER, _H, _S))
    b1_ref[...] = jnp.broadcast_to(fc1b_ref[...], (_INNER, _S))
    b2_ref[...] = jnp.broadcast_to(fc2b_ref[...], (_H, _S))
    ssum_ref[...] = jnp.zeros((_H, _S), jnp.float32)
    cnt_ref[...] = jnp.zeros((1, _S), jnp.float32)
    wsum_ref[...] = jnp.zeros((1, _S), jnp.float32)

    def gather(ids):
        iota_v = lax.broadcasted_iota(jnp.int32, (_VOCAB, _S), 0)
        oh = (iota_v == ids).astype(jnp.float32)
        g = jnp.dot(tab_ref[...], oh, preferred_element_type=jnp.float32)
        return g[0:_H] + g[_H:2 * _H] + g[2 * _H:3 * _H]

    def mlp_fwd(w1, b1, w2t, b2, k):
        z1 = jnp.sum(w1 * k[None, :, :], axis=1) + b1
        a = jnp.maximum(z1, 0.0)
        pred = jnp.sum(w2t * a[:, None, :], axis=0) + b2
        return z1, a, pred

    def outer(jo, _):
        rows = seq_ref[pl.ds(_UNROLL * 2 * jo, _UNROLL * 2), :]
        w1 = w1_ref[...]
        w2t = w2t_ref[...]
        b1 = b1_ref[...]
        b2 = b2_ref[...]
        ssum = ssum_ref[...]
        cnt = cnt_ref[...]
        wsum = wsum_ref[...]
        for slot in range(_UNROLL):
            step = _UNROLL * jo + slot
            k = gather(rows[2 * slot:2 * slot + 1, :])
            v = gather(rows[2 * slot + 1:2 * slot + 2, :])
            mean_h = ssum / jnp.maximum(cnt, 1.0)
            diff = k - mean_h
            dist = jnp.sqrt(jnp.sum(diff * diff, axis=0, keepdims=True))
            write = jnp.logical_or(cnt == 0.0, dist > _THR)
            write = jnp.logical_and(write, step < n_steps)
            wf = write.astype(jnp.float32)
            z1, a, pred = mlp_fwd(w1, b1, w2t, b2, k)
            dpred = (2.0 / _H) * (pred - v)
            dz1 = jnp.sum(w2t * dpred[None, :, :], axis=1)
            dz1 = dz1 * (z1 > 0).astype(jnp.float32)
            g = wf * _LR
            gdz1 = dz1 * g
            gdpred = dpred * g
            w1 = w1 - gdz1[:, None, :] * k[None, :, :]
            b1 = b1 - gdz1
            w2t = w2t - a[:, None, :] * gdpred[None, :, :]
            b2 = b2 - gdpred
            ssum = ssum + k
            cnt = cnt + 1.0
            wsum = wsum + wf
        w1_ref[...] = w1
        w2t_ref[...] = w2t
        b1_ref[...] = b1
        b2_ref[...] = b2
        ssum_ref[...] = ssum
        cnt_ref[...] = cnt
        wsum_ref[...] = wsum
        return ()

    n_outer = (n_steps + _UNROLL - 1) // _UNROLL
    lax.fori_loop(0, n_outer, outer, ())

    L = seq_ref.shape[0]
    last = seq_ref[L - 8:L, :]
    q = gather(last[7:8, :])
    _, aq, ctx = mlp_fwd(w1_ref[...], b1_ref[...], w2t_ref[...], b2_ref[...], q)
    logits_ref[...] = jnp.dot(outw_ref[...], ctx,
                              preferred_element_type=jnp.float32) + outb_ref[...]
    wcnt_ref[...] = wsum_ref[...][None, :, :]


def kernel(seq, embed, ff1_w, ff1_b, ff2_w, ff2_b, ln_g, ln_b,
           fc1_w, fc1_b, fc2_w, fc2_b, out_w, out_b, *, interpret=False):
    B, L = seq.shape
    n = (L - 3 + 1) // 2
    nblk = B // _S
    seq_T = seq.T.astype(jnp.int32)

    logits_T, wcnt = pl.pallas_call(
        functools.partial(_fw_kernel, n_steps=n),
        out_shape=[
            jax.ShapeDtypeStruct((_VOCAB, B), jnp.float32),
            jax.ShapeDtypeStruct((nblk, 1, _S), jnp.float32),
        ],
        grid=(nblk,),
        in_specs=[
            pl.BlockSpec((L, _S), lambda i: (0, i)),
            pl.BlockSpec((_H, _VOCAB), lambda i: (0, 0)),
            pl.BlockSpec((2 * _H, _H), lambda i: (0, 0)),
            pl.BlockSpec((2 * _H, 1), lambda i: (0, 0)),
            pl.BlockSpec((_H, 2 * _H), lambda i: (0, 0)),
            pl.BlockSpec((_H, 1), lambda i: (0, 0)),
            pl.BlockSpec((_H, 1), lambda i: (0, 0)),
            pl.BlockSpec((_H, 1), lambda i: (0, 0)),
            pl.BlockSpec((_INNER, _H), lambda i: (0, 0)),
            pl.BlockSpec((_INNER, 1), lambda i: (0, 0)),
            pl.BlockSpec((_INNER, _H), lambda i: (0, 0)),
            pl.BlockSpec((_H, 1), lambda i: (0, 0)),
            pl.BlockSpec((_VOCAB, _H), lambda i: (0, 0)),
            pl.BlockSpec((_VOCAB, 1), lambda i: (0, 0)),
        ],
        out_specs=[
            pl.BlockSpec((_VOCAB, _S), lambda i: (0, i)),
            pl.BlockSpec((1, 1, _S), lambda i: (i, 0, 0)),
        ],
        scratch_shapes=[
            pltpu.VMEM((3 * _H, _VOCAB), jnp.float32),
            pltpu.VMEM((_INNER, _H, _S), jnp.float32),
            pltpu.VMEM((_INNER, _H, _S), jnp.float32),
            pltpu.VMEM((_INNER, _S), jnp.float32),
            pltpu.VMEM((_H, _S), jnp.float32),
            pltpu.VMEM((_H, _S), jnp.float32),
            pltpu.VMEM((1, _S), jnp.float32),
            pltpu.VMEM((1, _S), jnp.float32),
        ],
        compiler_params=pltpu.CompilerParams(
            dimension_semantics=("arbitrary",),
        ),
        name="surprise_gated_fw",
        interpret=interpret,
    )(seq_T, embed.T, ff1_w, ff1_b[:, None], ff2_w, ff2_b[:, None],
      ln_g[:, None], ln_b[:, None], fc1_w, fc1_b[:, None], fc2_w.T,
      fc2_b[:, None], out_w, out_b[:, None])

    logits = logits_T.T
    write_rate = jnp.sum(wcnt) / jnp.float32(B * n)
    return logits, write_rate


# speedup vs baseline: 42.8119x; 1.0268x over previous
import functools

import jax
import jax.numpy as jnp
from jax import lax
from jax.experimental import pallas as pl
from jax.experimental.pallas import tpu as pltpu

_H = 32
_INNER = 8
_VOCAB = 64
_LR = 0.01
_THR = 4.0
_EPS = 1e-5
_S = 256
_UNROLL = 16


def _split3(x):
    hi = x.astype(jnp.bfloat16).astype(jnp.float32)
    r = x - hi
    mid = r.astype(jnp.bfloat16).astype(jnp.float32)
    lo = r - mid
    return hi, mid, lo


def _fw_kernel(seq_ref, embedT_ref, ff1w_ref, ff1b_ref, ff2w_ref, ff2b_ref,
               lng_ref, lnb_ref, fc1w_ref, fc1b_ref, fc2wT_ref, fc2b_ref,
               outw_ref, outb_ref,
               logits_ref, wcnt_ref,
               tab_ref, w1_ref, w2t_ref, b1_ref, b2_ref,
               ssum_ref, cnt_ref, wsum_ref, *, n_steps):
    eT = embedT_ref[...]
    zT = jnp.maximum(
        jnp.dot(ff1w_ref[...], eT, preferred_element_type=jnp.float32)
        + ff1b_ref[...], 0.0)
    fT = jnp.dot(ff2w_ref[...], zT, preferred_element_type=jnp.float32) \
        + ff2b_ref[...]
    xT = eT + fT
    mu = jnp.mean(xT, axis=0, keepdims=True)
    var = jnp.mean((xT - mu) ** 2, axis=0, keepdims=True)
    tabT = lng_ref[...] * (xT - mu) * lax.rsqrt(var + _EPS) + lnb_ref[...]
    hi, mid, lo = _split3(tabT)
    tab_ref[...] = jnp.concatenate([hi, mid, lo], axis=0)

    w1_ref[...] = jnp.broadcast_to(fc1w_ref[...][:, :, None], (_INNER, _H, _S))
    w2t_ref[...] = jnp.broadcast_to(fc2wT_ref[...][:, :, None], (_INNER, _H, _S))
    b1_ref[...] = jnp.broadcast_to(fc1b_ref[...], (_INNER, _S))
    b2_ref[...] = jnp.broadcast_to(fc2b_ref[...], (_H, _S))
    ssum_ref[...] = jnp.zeros((_H, _S), jnp.float32)
    cnt_ref[...] = jnp.zeros((1, _S), jnp.float32)
    wsum_ref[...] = jnp.zeros((1, _S), jnp.float32)

    def gather(ids):
        iota_v = lax.broadcasted_iota(jnp.int32, (_VOCAB, _S), 0)
        oh = (iota_v == ids).astype(jnp.float32)
        g = jnp.dot(tab_ref[...], oh, preferred_element_type=jnp.float32)
        return g[0:_H] + g[_H:2 * _H] + g[2 * _H:3 * _H]

    def mlp_fwd(w1, b1, w2t, b2, k):
        z1 = jnp.sum(w1 * k[None, :, :], axis=1) + b1
        a = jnp.maximum(z1, 0.0)
        pred = jnp.sum(w2t * a[:, None, :], axis=0) + b2
        return z1, a, pred

    def outer(jo, _):
        rows = seq_ref[pl.ds(_UNROLL * 2 * jo, _UNROLL * 2), :]
        w1 = w1_ref[...]
        w2t = w2t_ref[...]
        b1 = b1_ref[...]
        b2 = b2_ref[...]
        ssum = ssum_ref[...]
        cnt = cnt_ref[...]
        wsum = wsum_ref[...]
        for slot in range(_UNROLL):
            step = _UNROLL * jo + slot
            k = gather(rows[2 * slot:2 * slot + 1, :])
            v = gather(rows[2 * slot + 1:2 * slot + 2, :])
            mean_h = ssum / jnp.maximum(cnt, 1.0)
            diff = k - mean_h
            dist = jnp.sqrt(jnp.sum(diff * diff, axis=0, keepdims=True))
            write = jnp.logical_or(cnt == 0.0, dist > _THR)
            write = jnp.logical_and(write, step < n_steps)
            wf = write.astype(jnp.float32)
            z1, a, pred = mlp_fwd(w1, b1, w2t, b2, k)
            dpred = (2.0 / _H) * (pred - v)
            dz1 = jnp.sum(w2t * dpred[None, :, :], axis=1)
            dz1 = dz1 * (z1 > 0).astype(jnp.float32)
            g = wf * _LR
            gdz1 = dz1 * g
            gdpred = dpred * g
            w1 = w1 - gdz1[:, None, :] * k[None, :, :]
            b1 = b1 - gdz1
            w2t = w2t - a[:, None, :] * gdpred[None, :, :]
            b2 = b2 - gdpred
            ssum = ssum + k
            cnt = cnt + 1.0
            wsum = wsum + wf
        w1_ref[...] = w1
        w2t_ref[...] = w2t
        b1_ref[...] = b1
        b2_ref[...] = b2
        ssum_ref[...] = ssum
        cnt_ref[...] = cnt
        wsum_ref[...] = wsum
        return ()

    n_outer = (n_steps + _UNROLL - 1) // _UNROLL
    lax.fori_loop(0, n_outer, outer, ())

    L = seq_ref.shape[0]
    last = seq_ref[L - 8:L, :]
    q = gather(last[7:8, :])
    _, aq, ctx = mlp_fwd(w1_ref[...], b1_ref[...], w2t_ref[...], b2_ref[...], q)
    logits_ref[...] = jnp.dot(outw_ref[...], ctx,
                              preferred_element_type=jnp.float32) + outb_ref[...]
    wcnt_ref[...] = wsum_ref[...][None, :, :]


def kernel(seq, embed, ff1_w, ff1_b, ff2_w, ff2_b, ln_g, ln_b,
           fc1_w, fc1_b, fc2_w, fc2_b, out_w, out_b, *, interpret=False):
    B, L = seq.shape
    n = (L - 3 + 1) // 2
    nblk = B // _S
    seq_T = seq.T.astype(jnp.int32)

    logits_T, wcnt = pl.pallas_call(
        functools.partial(_fw_kernel, n_steps=n),
        out_shape=[
            jax.ShapeDtypeStruct((_VOCAB, B), jnp.float32),
            jax.ShapeDtypeStruct((nblk, 1, _S), jnp.float32),
        ],
        grid=(nblk,),
        in_specs=[
            pl.BlockSpec((L, _S), lambda i: (0, i)),
            pl.BlockSpec((_H, _VOCAB), lambda i: (0, 0)),
            pl.BlockSpec((2 * _H, _H), lambda i: (0, 0)),
            pl.BlockSpec((2 * _H, 1), lambda i: (0, 0)),
            pl.BlockSpec((_H, 2 * _H), lambda i: (0, 0)),
            pl.BlockSpec((_H, 1), lambda i: (0, 0)),
            pl.BlockSpec((_H, 1), lambda i: (0, 0)),
            pl.BlockSpec((_H, 1), lambda i: (0, 0)),
            pl.BlockSpec((_INNER, _H), lambda i: (0, 0)),
            pl.BlockSpec((_INNER, 1), lambda i: (0, 0)),
            pl.BlockSpec((_INNER, _H), lambda i: (0, 0)),
            pl.BlockSpec((_H, 1), lambda i: (0, 0)),
            pl.BlockSpec((_VOCAB, _H), lambda i: (0, 0)),
            pl.BlockSpec((_VOCAB, 1), lambda i: (0, 0)),
        ],
        out_specs=[
            pl.BlockSpec((_VOCAB, _S), lambda i: (0, i)),
            pl.BlockSpec((1, 1, _S), lambda i: (i, 0, 0)),
        ],
        scratch_shapes=[
            pltpu.VMEM((3 * _H, _VOCAB), jnp.float32),
            pltpu.VMEM((_INNER, _H, _S), jnp.float32),
            pltpu.VMEM((_INNER, _H, _S), jnp.float32),
            pltpu.VMEM((_INNER, _S), jnp.float32),
            pltpu.VMEM((_H, _S), jnp.float32),
            pltpu.VMEM((_H, _S), jnp.float32),
            pltpu.VMEM((1, _S), jnp.float32),
            pltpu.VMEM((1, _S), jnp.float32),
        ],
        compiler_params=pltpu.CompilerParams(
            dimension_semantics=("arbitrary",),
        ),
        name="surprise_gated_fw",
        interpret=interpret,
    )(seq_T, embed.T, ff1_w, ff1_b[:, None], ff2_w, ff2_b[:, None],
      ln_g[:, None], ln_b[:, None], fc1_w, fc1_b[:, None], fc2_w.T,
      fc2_b[:, None], out_w, out_b[:, None])

    logits = logits_T.T
    write_rate = jnp.sum(wcnt) / jnp.float32(B * n)
    return logits, write_rate
